# baseline (speedup 1.0000x reference)
import jax
import jax.numpy as jnp
from jax import lax
from jax.experimental import pallas as pl
from jax.experimental.pallas import tpu as pltpu

_B = 8192
_D = 768
_H = 256
_C = 2
_E = 8
_GH = 128
_BT = 512
_NS = _B // _BT
_W = _E * _H


def _moe_body(x_ref, gw1_ref, gb1_ref, gw2_ref, gb2_ref,
              We1_ref, be1_ref,
              scores_ref, idx_ref, p_ref, r_ref,
              wall_ref, ball_ref):
    step = pl.program_id(0)

    @pl.when(jnp.logical_or(step == 0, step == _NS // 2))
    def _init():
        for e in range(_E):
            lo = e * _H
            wall_ref[:, lo:lo + _H] = We1_ref[e].astype(jnp.bfloat16)
            ball_ref[:, lo:lo + _H] = be1_ref[e:e + 1, :]

    xb = x_ref[...]
    xb16 = xb.astype(jnp.bfloat16)
    hall = jnp.dot(xb16, wall_ref[...], preferred_element_type=jnp.float32)
    hall = hall + ball_ref[...]

    g1 = jnp.dot(xb, gw1_ref[...], preferred_element_type=jnp.float32)
    g1 = jnp.maximum(g1 + gb1_ref[...], 0.0)
    logits = jnp.dot(g1, gw2_ref[...], preferred_element_type=jnp.float32)
    logits = logits + gb2_ref[...]
    mx = jnp.max(logits, axis=-1, keepdims=True)
    ex = jnp.exp(logits - mx)
    scores = ex / jnp.sum(ex, axis=-1, keepdims=True)
    scores_ref[...] = scores

    eiota = lax.broadcasted_iota(jnp.int32, (_BT, _E), 1)
    m1 = jnp.max(scores, axis=-1, keepdims=True)
    i1 = jnp.min(jnp.where(scores == m1, eiota, _E), axis=-1, keepdims=True)
    masked = jnp.where(eiota == i1, -1.0, scores)
    m2 = jnp.max(masked, axis=-1, keepdims=True)
    i2 = jnp.min(jnp.where(masked == m2, eiota, _E), axis=-1, keepdims=True)
    idx_ref[...] = jnp.concatenate([i1, i2], axis=1)
    rd = 1.0 / (m1 + m2)
    w1 = m1 * rd
    w2 = m2 * rd
    gates = jnp.where(eiota == i1, w1, 0.0) + jnp.where(eiota == i2, w2, 0.0)
    r_ref[...] = jnp.sum(gates, axis=0, keepdims=True)[None]

    for e in range(_E):
        lo = e * _H
        h = hall[:, lo:lo + _H]
        h = 0.5 * h * (1.0 + lax.erf(h * 0.70710678118654752))
        mu = jnp.mean(h, axis=-1, keepdims=True)
        cen = h - mu
        var = jnp.mean(cen * cen, axis=-1, keepdims=True)
        ge = gates[:, e:e + 1]
        a = ge * lax.rsqrt(var + 1e-5)
        p_ref[0, e:e + 1, :] = jnp.sum(a * cen, axis=0, keepdims=True)


def _combine_body(p_ref, r_ref, ln_g_ref, ln_b_ref, We2_ref, be2_ref,
                  out8_ref):
    p = jnp.sum(p_ref[...], axis=0)
    r = jnp.sum(r_ref[...], axis=0)
    for e in range(_E):
        rb = jnp.broadcast_to(r[0:1, e:e + 1], (1, _H))
        s = ln_g_ref[e:e + 1, :] * p[e:e + 1, :] + ln_b_ref[e:e + 1, :] * rb
        o = jnp.dot(s, We2_ref[e], preferred_element_type=jnp.float32)
        out8_ref[e:e + 1, :] = o + be2_ref[e:e + 1, :] * rb[:, :_C]


def kernel(x, gw1, gb1, gw2, gb2, We1, be1, ln_g, ln_b, We2, be2):
    full = lambda i: (0, 0)
    full3 = lambda i: (0, 0, 0)
    scores, idx, p_part, r_part = pl.pallas_call(
        _moe_body,
        grid=(_NS,),
        in_specs=[
            pl.BlockSpec((_BT, _D), lambda i: (i, 0)),
            pl.BlockSpec((_D, _GH), full),
            pl.BlockSpec((1, _GH), full),
            pl.BlockSpec((_GH, _E), full),
            pl.BlockSpec((1, _E), full),
            pl.BlockSpec((_E, _D, _H), full3),
            pl.BlockSpec((_E, _H), full),
        ],
        out_specs=[
            pl.BlockSpec((_BT, _E), lambda i: (i, 0)),
            pl.BlockSpec((_BT, 2), lambda i: (i, 0)),
            pl.BlockSpec((1, _E, _H), lambda i: (i, 0, 0)),
            pl.BlockSpec((1, 1, _E), lambda i: (i, 0, 0)),
        ],
        out_shape=[
            jax.ShapeDtypeStruct((_B, _E), jnp.float32),
            jax.ShapeDtypeStruct((_B, 2), jnp.int32),
            jax.ShapeDtypeStruct((_NS, _E, _H), jnp.float32),
            jax.ShapeDtypeStruct((_NS, 1, _E), jnp.float32),
        ],
        scratch_shapes=[
            pltpu.VMEM((_D, _W), jnp.bfloat16),
            pltpu.VMEM((1, _W), jnp.float32),
        ],
        compiler_params=pltpu.CompilerParams(
            dimension_semantics=("parallel",)),
    )(x, gw1, gb1.reshape(1, _GH), gw2, gb2.reshape(1, _E), We1, be1)

    f2 = lambda: (0, 0)
    f3 = lambda: (0, 0, 0)
    out8 = pl.pallas_call(
        _combine_body,
        in_specs=[
            pl.BlockSpec((_NS, _E, _H), f3),
            pl.BlockSpec((_NS, 1, _E), f3),
            pl.BlockSpec((_E, _H), f2),
            pl.BlockSpec((_E, _H), f2),
            pl.BlockSpec((_E, _H, _C), f3),
            pl.BlockSpec((_E, _C), f2),
        ],
        out_specs=pl.BlockSpec((_E, _C), f2),
        out_shape=jax.ShapeDtypeStruct((_E, _C), jnp.float32),
    )(p_part, r_part, ln_g, ln_b, We2, be2)

    output = jnp.zeros((_B, _C), jnp.float32).at[:_E, :].set(out8)
    return output, scores, idx


# speedup vs baseline: 1.1140x; 1.1140x over previous
import jax
import jax.numpy as jnp
from jax import lax
from jax.experimental import pallas as pl
from jax.experimental.pallas import tpu as pltpu

_B = 8192
_D = 768
_H = 256
_C = 2
_E = 8
_GH = 128
_BT = 512
_NS = _B // _BT
_W = _E * _H


def _moe_body(x_ref, gw1_ref, gb1_ref, gw2_ref, gb2_ref,
              We1_ref, be1_ref,
              scores_ref, idx_ref, p_ref, r_ref,
              wall_ref, ball_ref, m1_ref):
    step = pl.program_id(0)

    @pl.when(jnp.logical_or(step == 0, step == _NS // 2))
    def _init():
        for e in range(_E):
            lo = e * _H
            wall_ref[:, lo:lo + _H] = We1_ref[e].astype(jnp.bfloat16)
            ball_ref[:, lo:lo + _H] = be1_ref[e:e + 1, :]
        seg = lax.broadcasted_iota(jnp.int32, (_W, _E), 0) // _H
        m1_ref[...] = jnp.where(
            seg == lax.broadcasted_iota(jnp.int32, (_W, _E), 1), 1.0, 0.0)

    xb = x_ref[...]
    xb16 = xb.astype(jnp.bfloat16)
    hall = jnp.dot(xb16, wall_ref[...], preferred_element_type=jnp.float32)
    hall = hall + ball_ref[...]

    g1 = jnp.dot(xb, gw1_ref[...], preferred_element_type=jnp.float32)
    g1 = jnp.maximum(g1 + gb1_ref[...], 0.0)
    logits = jnp.dot(g1, gw2_ref[...], preferred_element_type=jnp.float32)
    logits = logits + gb2_ref[...]
    mx = jnp.max(logits, axis=-1, keepdims=True)
    ex = jnp.exp(logits - mx)
    scores = ex / jnp.sum(ex, axis=-1, keepdims=True)
    scores_ref[...] = scores

    eiota = lax.broadcasted_iota(jnp.int32, (_BT, _E), 1)
    m1 = jnp.max(scores, axis=-1, keepdims=True)
    i1 = jnp.min(jnp.where(scores == m1, eiota, _E), axis=-1, keepdims=True)
    masked = jnp.where(eiota == i1, -1.0, scores)
    m2 = jnp.max(masked, axis=-1, keepdims=True)
    i2 = jnp.min(jnp.where(masked == m2, eiota, _E), axis=-1, keepdims=True)
    idx_ref[...] = jnp.concatenate([i1, i2], axis=1)
    rd = 1.0 / (m1 + m2)
    w1 = m1 * rd
    w2 = m2 * rd
    gates = jnp.where(eiota == i1, w1, 0.0) + jnp.where(eiota == i2, w2, 0.0)
    r_ref[...] = jnp.sum(gates, axis=0, keepdims=True)[None]

    g = 0.5 * hall * (1.0 + lax.erf(hall * 0.70710678118654752))
    sh = jnp.dot(g, m1_ref[...], preferred_element_type=jnp.float32)
    shh = jnp.dot(g * g, m1_ref[...], preferred_element_type=jnp.float32)
    mu8 = sh * (1.0 / _H)
    var8 = shh * (1.0 / _H) - mu8 * mu8
    amat = gates * lax.rsqrt(var8 + 1e-5)
    p1 = lax.dot_general(amat, g, (((0,), (0,)), ((), ())),
                         preferred_element_type=jnp.float32)
    q8 = jnp.sum(amat * mu8, axis=0, keepdims=True)
    for e in range(_E):
        lo = e * _H
        qb = jnp.broadcast_to(q8[0:1, e:e + 1], (1, _H))
        p_ref[0, e:e + 1, :] = p1[e:e + 1, lo:lo + _H] - qb


def _combine_body(p_ref, r_ref, ln_g_ref, ln_b_ref, We2_ref, be2_ref,
                  out8_ref):
    p = jnp.sum(p_ref[...], axis=0)
    r = jnp.sum(r_ref[...], axis=0)
    for e in range(_E):
        rb = jnp.broadcast_to(r[0:1, e:e + 1], (1, _H))
        s = ln_g_ref[e:e + 1, :] * p[e:e + 1, :] + ln_b_ref[e:e + 1, :] * rb
        o = jnp.dot(s, We2_ref[e], preferred_element_type=jnp.float32)
        out8_ref[e:e + 1, :] = o + be2_ref[e:e + 1, :] * rb[:, :_C]


def kernel(x, gw1, gb1, gw2, gb2, We1, be1, ln_g, ln_b, We2, be2):
    full = lambda i: (0, 0)
    full3 = lambda i: (0, 0, 0)
    scores, idx, p_part, r_part = pl.pallas_call(
        _moe_body,
        grid=(_NS,),
        in_specs=[
            pl.BlockSpec((_BT, _D), lambda i: (i, 0)),
            pl.BlockSpec((_D, _GH), full),
            pl.BlockSpec((1, _GH), full),
            pl.BlockSpec((_GH, _E), full),
            pl.BlockSpec((1, _E), full),
            pl.BlockSpec((_E, _D, _H), full3),
            pl.BlockSpec((_E, _H), full),
        ],
        out_specs=[
            pl.BlockSpec((_BT, _E), lambda i: (i, 0)),
            pl.BlockSpec((_BT, 2), lambda i: (i, 0)),
            pl.BlockSpec((1, _E, _H), lambda i: (i, 0, 0)),
            pl.BlockSpec((1, 1, _E), lambda i: (i, 0, 0)),
        ],
        out_shape=[
            jax.ShapeDtypeStruct((_B, _E), jnp.float32),
            jax.ShapeDtypeStruct((_B, 2), jnp.int32),
            jax.ShapeDtypeStruct((_NS, _E, _H), jnp.float32),
            jax.ShapeDtypeStruct((_NS, 1, _E), jnp.float32),
        ],
        scratch_shapes=[
            pltpu.VMEM((_D, _W), jnp.bfloat16),
            pltpu.VMEM((1, _W), jnp.float32),
            pltpu.VMEM((_W, _E), jnp.float32),
        ],
        compiler_params=pltpu.CompilerParams(
            dimension_semantics=("parallel",)),
    )(x, gw1, gb1.reshape(1, _GH), gw2, gb2.reshape(1, _E), We1, be1)

    f2 = lambda: (0, 0)
    f3 = lambda: (0, 0, 0)
    out8 = pl.pallas_call(
        _combine_body,
        in_specs=[
            pl.BlockSpec((_NS, _E, _H), f3),
            pl.BlockSpec((_NS, 1, _E), f3),
            pl.BlockSpec((_E, _H), f2),
            pl.BlockSpec((_E, _H), f2),
            pl.BlockSpec((_E, _H, _C), f3),
            pl.BlockSpec((_E, _C), f2),
        ],
        out_specs=pl.BlockSpec((_E, _C), f2),
        out_shape=jax.ShapeDtypeStruct((_E, _C), jnp.float32),
    )(p_part, r_part, ln_g, ln_b, We2, be2)

    output = jnp.zeros((_B, _C), jnp.float32).at[:_E, :].set(out8)
    return output, scores, idx


# speedup vs baseline: 1.1219x; 1.0071x over previous
import jax
import jax.numpy as jnp
from jax import lax
from jax.experimental import pallas as pl
from jax.experimental.pallas import tpu as pltpu

_B = 8192
_D = 768
_H = 256
_C = 2
_E = 8
_GH = 128
_BT = 512
_NS = _B // _BT
_W = _E * _H


def _moe_body(x_ref, gw1_ref, gb1_ref, gw2_ref, gb2_ref,
              We1_ref, be1_ref, ln_g_ref, ln_b_ref, We2_ref, be2_ref,
              scores_ref, idx_ref, out8_ref,
              wall_ref, ball_ref, m1_ref, p_acc, q_acc, r_acc):
    step = pl.program_id(0)

    @pl.when(step == 0)
    def _init():
        for e in range(_E):
            lo = e * _H
            wall_ref[:, lo:lo + _H] = We1_ref[e].astype(jnp.bfloat16)
            ball_ref[:, lo:lo + _H] = be1_ref[e:e + 1, :]
        seg = lax.broadcasted_iota(jnp.int32, (_W, _E), 0) // _H
        m1_ref[...] = jnp.where(
            seg == lax.broadcasted_iota(jnp.int32, (_W, _E), 1), 1.0, 0.0)
        p_acc[...] = jnp.zeros_like(p_acc)
        q_acc[...] = jnp.zeros_like(q_acc)
        r_acc[...] = jnp.zeros_like(r_acc)

    xb = x_ref[...]
    xb16 = xb.astype(jnp.bfloat16)
    hall = jnp.dot(xb16, wall_ref[...], preferred_element_type=jnp.float32)
    hall = hall + ball_ref[...]

    g1 = jnp.dot(xb, gw1_ref[...], preferred_element_type=jnp.float32)
    g1 = jnp.maximum(g1 + gb1_ref[...], 0.0)
    logits = jnp.dot(g1, gw2_ref[...], preferred_element_type=jnp.float32)
    logits = logits + gb2_ref[...]
    mx = jnp.max(logits, axis=-1, keepdims=True)
    ex = jnp.exp(logits - mx)
    scores = ex / jnp.sum(ex, axis=-1, keepdims=True)
    scores_ref[...] = scores

    eiota = lax.broadcasted_iota(jnp.int32, (_BT, _E), 1)
    m1 = jnp.max(scores, axis=-1, keepdims=True)
    i1 = jnp.min(jnp.where(scores == m1, eiota, _E), axis=-1, keepdims=True)
    masked = jnp.where(eiota == i1, -1.0, scores)
    m2 = jnp.max(masked, axis=-1, keepdims=True)
    i2 = jnp.min(jnp.where(masked == m2, eiota, _E), axis=-1, keepdims=True)
    idx_ref[...] = jnp.concatenate([i1, i2], axis=1)
    rd = 1.0 / (m1 + m2)
    w1 = m1 * rd
    w2 = m2 * rd
    gates = jnp.where(eiota == i1, w1, 0.0) + jnp.where(eiota == i2, w2, 0.0)
    r_acc[...] += jnp.sum(gates, axis=0, keepdims=True)

    g = hall + hall * lax.erf(hall * 0.70710678118654752)
    sh = jnp.dot(g, m1_ref[...], preferred_element_type=jnp.float32)
    shh = jnp.dot(g * g, m1_ref[...], preferred_element_type=jnp.float32)
    mu8 = sh * (1.0 / _H)
    var8 = shh * (1.0 / _H) - mu8 * mu8
    amat = gates * lax.rsqrt(var8 + 4e-5)
    p1 = lax.dot_general(amat, g, (((0,), (0,)), ((), ())),
                         preferred_element_type=jnp.float32)
    p_acc[...] += p1
    q_acc[...] += jnp.sum(amat * mu8, axis=0, keepdims=True)

    @pl.when(step == _NS - 1)
    def _finish():
        for e in range(_E):
            lo = e * _H
            qb = jnp.broadcast_to(q_acc[0:1, e:e + 1], (1, _H))
            rb = jnp.broadcast_to(r_acc[0:1, e:e + 1], (1, _H))
            s = (ln_g_ref[e:e + 1, :] * (p_acc[e:e + 1, lo:lo + _H] - qb)
                 + ln_b_ref[e:e + 1, :] * rb)
            o = jnp.dot(s, We2_ref[e], preferred_element_type=jnp.float32)
            out8_ref[e:e + 1, :] = o + be2_ref[e:e + 1, :] * rb[:, :_C]


def kernel(x, gw1, gb1, gw2, gb2, We1, be1, ln_g, ln_b, We2, be2):
    full = lambda i: (0, 0)
    full3 = lambda i: (0, 0, 0)
    scores, idx, out8 = pl.pallas_call(
        _moe_body,
        grid=(_NS,),
        in_specs=[
            pl.BlockSpec((_BT, _D), lambda i: (i, 0)),
            pl.BlockSpec((_D, _GH), full),
            pl.BlockSpec((1, _GH), full),
            pl.BlockSpec((_GH, _E), full),
            pl.BlockSpec((1, _E), full),
            pl.BlockSpec((_E, _D, _H), full3),
            pl.BlockSpec((_E, _H), full),
            pl.BlockSpec((_E, _H), full),
            pl.BlockSpec((_E, _H), full),
            pl.BlockSpec((_E, _H, _C), full3),
            pl.BlockSpec((_E, _C), full),
        ],
        out_specs=[
            pl.BlockSpec((_BT, _E), lambda i: (i, 0)),
            pl.BlockSpec((_BT, 2), lambda i: (i, 0)),
            pl.BlockSpec((_E, _C), full),
        ],
        out_shape=[
            jax.ShapeDtypeStruct((_B, _E), jnp.float32),
            jax.ShapeDtypeStruct((_B, 2), jnp.int32),
            jax.ShapeDtypeStruct((_E, _C), jnp.float32),
        ],
        scratch_shapes=[
            pltpu.VMEM((_D, _W), jnp.bfloat16),
            pltpu.VMEM((1, _W), jnp.float32),
            pltpu.VMEM((_W, _E), jnp.float32),
            pltpu.VMEM((_E, _W), jnp.float32),
            pltpu.VMEM((1, _E), jnp.float32),
            pltpu.VMEM((1, _E), jnp.float32),
        ],
    )(x, gw1, gb1.reshape(1, _GH), gw2, gb2.reshape(1, _E),
      We1, be1, ln_g, ln_b, We2, be2)
    output = jnp.zeros((_B, _C), jnp.float32).at[:_E, :].set(out8)
    return output, scores, idx


# speedup vs baseline: 1.1664x; 1.0396x over previous
import jax
import jax.numpy as jnp
from jax import lax
from jax.experimental import pallas as pl
from jax.experimental.pallas import tpu as pltpu

_B = 8192
_D = 768
_H = 256
_C = 2
_E = 8
_GH = 128
_BT = 1024
_NS = _B // _BT
_W = _E * _H


def _moe_body(x_ref, gw1_ref, gb1_ref, gw2_ref, gb2_ref,
              We1_ref, be1_ref, ln_g_ref, ln_b_ref, We2_ref, be2_ref,
              scores_ref, idx_ref, out8_ref,
              wall_ref, ball_ref, m1_ref, p_acc, q_acc, r_acc):
    step = pl.program_id(0)

    @pl.when(step == 0)
    def _init():
        for e in range(_E):
            lo = e * _H
            wall_ref[:, lo:lo + _H] = We1_ref[e].astype(jnp.bfloat16)
            ball_ref[:, lo:lo + _H] = be1_ref[e:e + 1, :]
        seg = lax.broadcasted_iota(jnp.int32, (_W, _E), 0) // _H
        m1_ref[...] = jnp.where(
            seg == lax.broadcasted_iota(jnp.int32, (_W, _E), 1), 1.0, 0.0)
        p_acc[...] = jnp.zeros_like(p_acc)
        q_acc[...] = jnp.zeros_like(q_acc)
        r_acc[...] = jnp.zeros_like(r_acc)

    xb = x_ref[...]
    xb16 = xb.astype(jnp.bfloat16)
    hall = jnp.dot(xb16, wall_ref[...], preferred_element_type=jnp.float32)
    hall = hall + ball_ref[...]

    g1 = jnp.dot(xb, gw1_ref[...], preferred_element_type=jnp.float32)
    g1 = jnp.maximum(g1 + gb1_ref[...], 0.0)
    logits = jnp.dot(g1, gw2_ref[...], preferred_element_type=jnp.float32)
    logits = logits + gb2_ref[...]
    mx = jnp.max(logits, axis=-1, keepdims=True)
    ex = jnp.exp(logits - mx)
    scores = ex / jnp.sum(ex, axis=-1, keepdims=True)
    scores_ref[...] = scores

    eiota = lax.broadcasted_iota(jnp.int32, (_BT, _E), 1)
    m1 = jnp.max(scores, axis=-1, keepdims=True)
    i1 = jnp.min(jnp.where(scores == m1, eiota, _E), axis=-1, keepdims=True)
    masked = jnp.where(eiota == i1, -1.0, scores)
    m2 = jnp.max(masked, axis=-1, keepdims=True)
    i2 = jnp.min(jnp.where(masked == m2, eiota, _E), axis=-1, keepdims=True)
    idx_ref[...] = jnp.concatenate([i1, i2], axis=1)
    rd = 1.0 / (m1 + m2)
    w1 = m1 * rd
    w2 = m2 * rd
    gates = jnp.where(eiota == i1, w1, 0.0) + jnp.where(eiota == i2, w2, 0.0)
    r_acc[...] += jnp.sum(gates, axis=0, keepdims=True)

    g = hall + hall * lax.erf(hall * 0.70710678118654752)
    sh = jnp.dot(g, m1_ref[...], preferred_element_type=jnp.float32)
    shh = jnp.dot(g * g, m1_ref[...], preferred_element_type=jnp.float32)
    mu8 = sh * (1.0 / _H)
    var8 = shh * (1.0 / _H) - mu8 * mu8
    amat = gates * lax.rsqrt(var8 + 4e-5)
    p1 = lax.dot_general(amat, g, (((0,), (0,)), ((), ())),
                         preferred_element_type=jnp.float32)
    p_acc[...] += p1
    q_acc[...] += jnp.sum(amat * mu8, axis=0, keepdims=True)

    @pl.when(step == _NS - 1)
    def _finish():
        for e in range(_E):
            lo = e * _H
            qb = jnp.broadcast_to(q_acc[0:1, e:e + 1], (1, _H))
            rb = jnp.broadcast_to(r_acc[0:1, e:e + 1], (1, _H))
            s = (ln_g_ref[e:e + 1, :] * (p_acc[e:e + 1, lo:lo + _H] - qb)
                 + ln_b_ref[e:e + 1, :] * rb)
            o = jnp.dot(s, We2_ref[e], preferred_element_type=jnp.float32)
            out8_ref[e:e + 1, :] = o + be2_ref[e:e + 1, :] * rb[:, :_C]


def kernel(x, gw1, gb1, gw2, gb2, We1, be1, ln_g, ln_b, We2, be2):
    full = lambda i: (0, 0)
    full3 = lambda i: (0, 0, 0)
    scores, idx, out8 = pl.pallas_call(
        _moe_body,
        grid=(_NS,),
        in_specs=[
            pl.BlockSpec((_BT, _D), lambda i: (i, 0)),
            pl.BlockSpec((_D, _GH), full),
            pl.BlockSpec((1, _GH), full),
            pl.BlockSpec((_GH, _E), full),
            pl.BlockSpec((1, _E), full),
            pl.BlockSpec((_E, _D, _H), full3),
            pl.BlockSpec((_E, _H), full),
            pl.BlockSpec((_E, _H), full),
            pl.BlockSpec((_E, _H), full),
            pl.BlockSpec((_E, _H, _C), full3),
            pl.BlockSpec((_E, _C), full),
        ],
        out_specs=[
            pl.BlockSpec((_BT, _E), lambda i: (i, 0)),
            pl.BlockSpec((_BT, 2), lambda i: (i, 0)),
            pl.BlockSpec((_E, _C), full),
        ],
        out_shape=[
            jax.ShapeDtypeStruct((_B, _E), jnp.float32),
            jax.ShapeDtypeStruct((_B, 2), jnp.int32),
            jax.ShapeDtypeStruct((_E, _C), jnp.float32),
        ],
        scratch_shapes=[
            pltpu.VMEM((_D, _W), jnp.bfloat16),
            pltpu.VMEM((1, _W), jnp.float32),
            pltpu.VMEM((_W, _E), jnp.float32),
            pltpu.VMEM((_E, _W), jnp.float32),
            pltpu.VMEM((1, _E), jnp.float32),
            pltpu.VMEM((1, _E), jnp.float32),
        ],
    )(x, gw1, gb1.reshape(1, _GH), gw2, gb2.reshape(1, _E),
      We1, be1, ln_g, ln_b, We2, be2)
    output = jnp.zeros((_B, _C), jnp.float32).at[:_E, :].set(out8)
    return output, scores, idx
